# baseline (speedup 1.0000x reference)
import jax
import jax.numpy as jnp
from jax.experimental import pallas as pl

ROW_BLOCK = 8192


def _copy_kernel(table_ref, out_ref):
    out_ref[...] = table_ref[...]


def kernel(x, table):
    seq_len = x.shape[1]
    embed_dim = table.shape[1]
    assert seq_len % ROW_BLOCK == 0
    grid = (seq_len // ROW_BLOCK,)
    return pl.pallas_call(
        _copy_kernel,
        grid=grid,
        in_specs=[pl.BlockSpec((ROW_BLOCK, embed_dim), lambda i: (i, 0))],
        out_specs=pl.BlockSpec((ROW_BLOCK, embed_dim), lambda i: (i, 0)),
        out_shape=jax.ShapeDtypeStruct((seq_len, embed_dim), table.dtype),
    )(table)


# speedup vs baseline: 1.0238x; 1.0238x over previous
import jax
import jax.numpy as jnp
from jax.experimental import pallas as pl
from jax.experimental.pallas import tpu as pltpu

NUM_CHUNKS = 8


def _dma_pipeline_kernel(table_ref, out_ref, buf, in_sems, out_sems):
    seq_len = out_ref.shape[0]
    chunk = seq_len // NUM_CHUNKS

    in_copies = []
    for i in range(NUM_CHUNKS):
        c = pltpu.make_async_copy(
            table_ref.at[pl.ds(i * chunk, chunk)],
            buf.at[pl.ds(i * chunk, chunk)],
            in_sems.at[i],
        )
        c.start()
        in_copies.append(c)

    out_copies = []
    for i in range(NUM_CHUNKS):
        in_copies[i].wait()
        c = pltpu.make_async_copy(
            buf.at[pl.ds(i * chunk, chunk)],
            out_ref.at[pl.ds(i * chunk, chunk)],
            out_sems.at[i],
        )
        c.start()
        out_copies.append(c)

    for c in out_copies:
        c.wait()


def kernel(x, table):
    seq_len = x.shape[1]
    embed_dim = table.shape[1]
    return pl.pallas_call(
        _dma_pipeline_kernel,
        in_specs=[pl.BlockSpec(memory_space=pltpu.HBM)],
        out_specs=pl.BlockSpec(memory_space=pltpu.HBM),
        out_shape=jax.ShapeDtypeStruct((seq_len, embed_dim), table.dtype),
        scratch_shapes=[
            pltpu.VMEM((seq_len, embed_dim), table.dtype),
            pltpu.SemaphoreType.DMA((NUM_CHUNKS,)),
            pltpu.SemaphoreType.DMA((NUM_CHUNKS,)),
        ],
    )(table)
